# baseline (speedup 1.0000x reference)
import functools

import jax
import jax.numpy as jnp
from jax import lax
from jax.experimental import pallas as pl
from jax.experimental.pallas import tpu as pltpu
from jax.experimental.pallas import tpu_sc as plsc

N = 10000
E = 320000
NPAD = 10240
EPAD = 32 * NPAD
G = 128
NB = 2
F32 = jnp.float32

_MESH = plsc.VectorSubcoreMesh(core_axis_name="c", subcore_axis_name="s")


def _make_sc_prop(feat, feat_split=False, const_ones=False, two_tables=False):
    nbatch = (EPAD // 16 if feat_split else EPAD // 32) // G
    CH = 32 if feat_split else 40
    wide = feat > 1
    acc_shape = (NPAD, feat) if wide else (NPAD,)
    row_shape = (G, feat) if wide else (G,)
    zrows = 32
    out_sd = jax.ShapeDtypeStruct(acc_shape, F32)

    per_core_tbl = feat_split or two_tables
    n_tables = 0 if const_ones else (2 if per_core_tbl else 1)
    scratch = [pltpu.VMEM_SHARED(acc_shape, F32)]
    scratch.append(pltpu.VMEM((zrows, feat) if wide else (640,), F32))
    if not const_ones:
        scratch.append(pltpu.VMEM((CH, G), jnp.int32))
    scratch.append(pltpu.VMEM((CH, G), jnp.int32))
    scratch += [pltpu.VMEM(row_shape, F32) for _ in range(NB)]
    scratch += [pltpu.SemaphoreType.DMA for _ in range(2 * NB)]

    @functools.partial(
        pl.kernel,
        out_type=(out_sd, out_sd),
        scratch_types=scratch,
        mesh=_MESH,
    )
    def prop(*refs):
        k = n_tables
        tables = refs[:k]
        if not const_ones:
            src2d_h = refs[k]; k += 1
        dst2d_h = refs[k]; k += 1
        out0, out1 = refs[k], refs[k + 1]; k += 2
        acc = refs[k]; k += 1
        zbuf = refs[k]; k += 1
        if not const_ones:
            srci = refs[k]; k += 1
        dsti = refs[k]; k += 1
        rows = refs[k:k + NB]; k += NB
        sems = refs[k:k + NB]; k += NB
        ssems = refs[k:k + NB]

        c = lax.axis_index("c")
        s = lax.axis_index("s")

        if wide:
            def zb(i, carry):
                zbuf[i // 8, pl.ds((i % 8) * 16, 16)] = jnp.zeros((16,), F32)
                return carry
            lax.fori_loop(0, zrows * 8, zb, 0)
            def zcp(kk, carry):
                pltpu.sync_copy(zbuf, acc.at[pl.ds(s * 640 + kk * zrows, zrows)])
                return carry
            lax.fori_loop(0, 640 // zrows, zcp, 0)
        else:
            def zb(i, carry):
                zbuf[pl.ds(i * 16, 16)] = jnp.zeros((16,), F32)
                return carry
            lax.fori_loop(0, 40, zb, 0)
            pltpu.sync_copy(zbuf, acc.at[pl.ds(s * 640, 640)])
        plsc.subcore_barrier()

        ibase = s * nbatch if feat_split else c * 16 * nbatch + s * nbatch

        if const_ones:
            for b in range(NB):
                def ob(i, carry, _b=b):
                    rows[_b][pl.ds(i * 16, 16)] = jnp.ones((16,), F32)
                    return carry
                lax.fori_loop(0, G // 16, ob, 0)

            def chunk(ci, carry):
                pltpu.sync_copy(dst2d_h.at[pl.ds(ibase + ci * CH, CH)], dsti)
                for b in range(NB):
                    pltpu.async_copy(rows[b], acc.at[dsti.at[b]], sems[b], add=True)
                def steady(j2, carry2):
                    for b in range(NB):
                        jj = j2 * NB + b
                        pltpu.make_async_copy(
                            rows[b], acc.at[dsti.at[jj - NB]], sems[b]).wait()
                        pltpu.async_copy(rows[b], acc.at[dsti.at[jj]], sems[b],
                                         add=True)
                    return carry2
                lax.fori_loop(1, CH // NB, steady, 0)
                for b in range(NB):
                    jj = CH - NB + b
                    pltpu.make_async_copy(rows[b], acc.at[dsti.at[jj]], sems[b]).wait()
                return carry
            lax.fori_loop(0, nbatch // CH, chunk, 0)
        else:
            def run(tbl):
                def chunk(ci, carry):
                    base = ibase + ci * CH
                    pltpu.sync_copy(src2d_h.at[pl.ds(base, CH)], srci)
                    pltpu.sync_copy(dst2d_h.at[pl.ds(base, CH)], dsti)
                    for b in range(NB):
                        pltpu.async_copy(tbl.at[srci.at[b]], rows[b], sems[b])
                    def steady(j2, carry2):
                        for b in range(NB):
                            jj = j2 * NB + b
                            pltpu.make_async_copy(
                                tbl.at[srci.at[jj]], rows[b], sems[b]).wait()
                            pltpu.sync_copy(rows[b], acc.at[dsti.at[jj]], add=True)
                            pltpu.async_copy(tbl.at[srci.at[jj + NB]], rows[b],
                                             sems[b])
                        return carry2
                    lax.fori_loop(0, CH // NB - 1, steady, 0)
                    for b in range(NB):
                        jj = CH - NB + b
                        pltpu.make_async_copy(
                            tbl.at[srci.at[jj]], rows[b], sems[b]).wait()
                        pltpu.sync_copy(rows[b], acc.at[dsti.at[jj]], add=True)
                    return carry
                lax.fori_loop(0, nbatch // CH, chunk, 0)

            if per_core_tbl:
                @pl.when(c == 0)
                def _():
                    run(tables[0])
                @pl.when(c == 1)
                def _():
                    run(tables[1])
            else:
                run(tables[0])

        plsc.subcore_barrier()

        @pl.when(c == 0)
        def _():
            pltpu.sync_copy(acc.at[pl.ds(s * 640, 640)], out0.at[pl.ds(s * 640, 640)])
        @pl.when(c == 1)
        def _():
            pltpu.sync_copy(acc.at[pl.ds(s * 640, 640)], out1.at[pl.ds(s * 640, 640)])

    return prop


_sc_deg = _make_sc_prop(1, const_ones=True)
_sc_prop1 = _make_sc_prop(1)
_sc_prop128 = _make_sc_prop(128, two_tables=True)
_sc_prop256 = _make_sc_prop(128, feat_split=True)


def _tca_body(d0, d1, x, dis_o, t0_o, t0b_o):
    deg = d0[...] + d1[...] + 1.0
    dis = lax.rsqrt(deg)
    dis_o[...] = dis
    t0 = x[...] * dis
    t0_o[...] = t0
    t0b_o[...] = t0
def _tc_a(d0, d1, x):
    R = 2000
    return pl.pallas_call(
        _tca_body,
        grid=(N // R,),
        in_specs=[
            pl.BlockSpec((R, 1), lambda i: (i, 0)),
            pl.BlockSpec((R, 1), lambda i: (i, 0)),
            pl.BlockSpec((R, 128), lambda i: (i, 0)),
        ],
        out_specs=[
            pl.BlockSpec((R, 1), lambda i: (i, 0)),
            pl.BlockSpec((R, 128), lambda i: (i, 0)),
            pl.BlockSpec((R, 128), lambda i: (i, 0)),
        ],
        out_shape=[
            jax.ShapeDtypeStruct((N, 1), F32),
            jax.ShapeDtypeStruct((N, 128), F32),
            jax.ShapeDtypeStruct((N, 128), F32),
        ],
    )(d0, d1, x)


def _tcb_body(a0, a1, t0, dis, W1, b1, o0, o1):
    d = dis[...]
    z = (a0[...] + a1[...] + t0[...]) * d
    h = jnp.maximum(jnp.dot(z, W1[...], preferred_element_type=F32) + b1[...], 0.0)
    t1 = h * d
    o0[...] = t1[:, :128]
    o1[...] = t1[:, 128:]


def _tc_b(a0, a1, t0, dis, W1, b1):
    R = 1000
    return pl.pallas_call(
        _tcb_body,
        grid=(N // R,),
        in_specs=[
            pl.BlockSpec((R, 128), lambda i: (i, 0)),
            pl.BlockSpec((R, 128), lambda i: (i, 0)),
            pl.BlockSpec((R, 128), lambda i: (i, 0)),
            pl.BlockSpec((R, 1), lambda i: (i, 0)),
            pl.BlockSpec((128, 256), lambda i: (0, 0)),
            pl.BlockSpec((1, 256), lambda i: (0, 0)),
        ],
        out_specs=[
            pl.BlockSpec((R, 128), lambda i: (i, 0)),
            pl.BlockSpec((R, 128), lambda i: (i, 0)),
        ],
        out_shape=[
            jax.ShapeDtypeStruct((N, 128), F32),
            jax.ShapeDtypeStruct((N, 128), F32),
        ],
    )(a0, a1, t0, dis, W1, b1)


def _tcc_body(c0, c1, t1a, t1b, dis, W2, b2, W3, o):
    d = dis[...]
    zA = (c0[...] + t1a[...]) * d
    zB = (c1[...] + t1b[...]) * d
    z = jnp.concatenate([zA, zB], axis=1)
    h = jnp.maximum(jnp.dot(z, W2[...], preferred_element_type=F32) + b2[...], 0.0)
    o[...] = jnp.dot(h, W3[...], preferred_element_type=F32) * d


def _tc_c(c0, c1, t1a, t1b, dis, W2, b2, W3):
    R = 1000
    return pl.pallas_call(
        _tcc_body,
        grid=(N // R,),
        in_specs=[
            pl.BlockSpec((R, 128), lambda i: (i, 0)),
            pl.BlockSpec((R, 128), lambda i: (i, 0)),
            pl.BlockSpec((R, 128), lambda i: (i, 0)),
            pl.BlockSpec((R, 128), lambda i: (i, 0)),
            pl.BlockSpec((R, 1), lambda i: (i, 0)),
            pl.BlockSpec((256, 512), lambda i: (0, 0)),
            pl.BlockSpec((1, 512), lambda i: (0, 0)),
            pl.BlockSpec((512, 1), lambda i: (0, 0)),
        ],
        out_specs=pl.BlockSpec((R, 1), lambda i: (i, 0)),
        out_shape=jax.ShapeDtypeStruct((N, 1), F32),
    )(c0, c1, t1a, t1b, dis, W2, b2, W3)


def _tcd_body(p0, p1, t3, dis, b3, o):
    z = (p0[...] + p1[...] + t3[...]) * dis[...] + b3[...]
    o[...] = 1.0 / (1.0 + jnp.exp(-z))


def _tc_d(p0, p1, t3, dis, b3):
    R = 2000
    return pl.pallas_call(
        _tcd_body,
        grid=(N // R,),
        in_specs=[
            pl.BlockSpec((R, 1), lambda i: (i, 0)),
            pl.BlockSpec((R, 1), lambda i: (i, 0)),
            pl.BlockSpec((R, 1), lambda i: (i, 0)),
            pl.BlockSpec((R, 1), lambda i: (i, 0)),
            pl.BlockSpec((1, 1), lambda i: (0, 0)),
        ],
        out_specs=pl.BlockSpec((R, 1), lambda i: (i, 0)),
        out_shape=jax.ShapeDtypeStruct((N, 1), F32),
    )(p0, p1, t3, dis, b3)


def kernel(x, edge_index, W1, b1, W2, b2, W3, b3):
    src = edge_index[0].astype(jnp.int32)
    dst = edge_index[1].astype(jnp.int32)
    npadrows = NPAD - N
    pad_dst = N + (jnp.arange(EPAD - E, dtype=jnp.int32) % npadrows)
    src2d = jnp.concatenate(
        [src, jnp.zeros((EPAD - E,), jnp.int32)]).reshape(EPAD // G, G)
    dst2d = jnp.concatenate([dst, pad_dst]).reshape(EPAD // G, G)

    d0, d1 = _sc_deg(dst2d)
    dis, t0, t0b = _tc_a(d0.reshape(NPAD, 1), d1.reshape(NPAD, 1), x)
    a0, a1 = _sc_prop128(t0, t0b, src2d, dst2d)
    t1a, t1b = _tc_b(a0, a1, t0, dis, W1, b1.reshape(1, -1))
    c0, c1 = _sc_prop256(t1a, t1b, src2d, dst2d)
    t3 = _tc_c(c0, c1, t1a, t1b, dis, W2, b2.reshape(1, -1), W3)
    p0, p1 = _sc_prop1(t3.reshape(N), src2d, dst2d)
    return _tc_d(p0.reshape(NPAD, 1), p1.reshape(NPAD, 1), t3, dis,
                 b3.reshape(1, 1))


# speedup vs baseline: 1.7743x; 1.7743x over previous
import functools

import jax
import jax.numpy as jnp
from jax import lax
from jax.experimental import pallas as pl
from jax.experimental.pallas import tpu as pltpu
from jax.experimental.pallas import tpu_sc as plsc

N = 10000
E = 320000
NPAD = 10240
EPAD = 32 * NPAD
G = 128
NB = 2
F32 = jnp.float32

_MESH = plsc.VectorSubcoreMesh(core_axis_name="c", subcore_axis_name="s")


def _make_sc_prop(feat, feat_split=False, const_ones=False, two_tables=False):
    nbatch = (EPAD // 16 if feat_split else EPAD // 32) // G
    CH = 32 if feat_split else 40
    wide = feat > 1
    acc_shape = (NPAD, feat) if wide else (NPAD,)
    row_shape = (G, feat) if wide else (G,)
    zrows = 32
    out_sd = jax.ShapeDtypeStruct(acc_shape, F32)

    per_core_tbl = feat_split or two_tables
    n_tables = 0 if const_ones else (2 if per_core_tbl else 1)
    scratch = [pltpu.VMEM_SHARED(acc_shape, F32)]
    scratch.append(pltpu.VMEM((zrows, feat) if wide else (640,), F32))
    if not const_ones:
        scratch.append(pltpu.VMEM((CH, G), jnp.int32))
    scratch.append(pltpu.VMEM((CH, G), jnp.int32))
    scratch += [pltpu.VMEM(row_shape, F32) for _ in range(NB)]
    scratch += [pltpu.SemaphoreType.DMA for _ in range(2 * NB)]

    @functools.partial(
        pl.kernel,
        out_type=(out_sd, out_sd),
        scratch_types=scratch,
        mesh=_MESH,
    )
    def prop(*refs):
        k = n_tables
        tables = refs[:k]
        if not const_ones:
            src2d_h = refs[k]; k += 1
        dst2d_h = refs[k]; k += 1
        out0, out1 = refs[k], refs[k + 1]; k += 2
        acc = refs[k]; k += 1
        zbuf = refs[k]; k += 1
        if not const_ones:
            srci = refs[k]; k += 1
        dsti = refs[k]; k += 1
        rows = refs[k:k + NB]; k += NB
        sems = refs[k:k + NB]; k += NB
        ssems = refs[k:k + NB]

        c = lax.axis_index("c")
        s = lax.axis_index("s")

        if wide:
            def zb(i, carry):
                zbuf[i // 8, pl.ds((i % 8) * 16, 16)] = jnp.zeros((16,), F32)
                return carry
            lax.fori_loop(0, zrows * 8, zb, 0)
            def zcp(kk, carry):
                pltpu.sync_copy(zbuf, acc.at[pl.ds(s * 640 + kk * zrows, zrows)])
                return carry
            lax.fori_loop(0, 640 // zrows, zcp, 0)
        else:
            def zb(i, carry):
                zbuf[pl.ds(i * 16, 16)] = jnp.zeros((16,), F32)
                return carry
            lax.fori_loop(0, 40, zb, 0)
            pltpu.sync_copy(zbuf, acc.at[pl.ds(s * 640, 640)])
        plsc.subcore_barrier()

        ibase = s * nbatch if feat_split else c * 16 * nbatch + s * nbatch

        if const_ones:
            for b in range(NB):
                def ob(i, carry, _b=b):
                    rows[_b][pl.ds(i * 16, 16)] = jnp.ones((16,), F32)
                    return carry
                lax.fori_loop(0, G // 16, ob, 0)

            def chunk(ci, carry):
                pltpu.sync_copy(dst2d_h.at[pl.ds(ibase + ci * CH, CH)], dsti)
                for b in range(NB):
                    pltpu.async_copy(rows[b], acc.at[dsti.at[b]], sems[b], add=True)
                def steady(j2, carry2):
                    for b in range(NB):
                        jj = j2 * NB + b
                        pltpu.make_async_copy(
                            rows[b], acc.at[dsti.at[jj - NB]], sems[b]).wait()
                        pltpu.async_copy(rows[b], acc.at[dsti.at[jj]], sems[b],
                                         add=True)
                    return carry2
                lax.fori_loop(1, CH // NB, steady, 0)
                for b in range(NB):
                    jj = CH - NB + b
                    pltpu.make_async_copy(rows[b], acc.at[dsti.at[jj]], sems[b]).wait()
                return carry
            lax.fori_loop(0, nbatch // CH, chunk, 0)
        else:
            def run(tbl):
                def chunk(ci, carry):
                    base = ibase + ci * CH
                    pltpu.sync_copy(src2d_h.at[pl.ds(base, CH)], srci)
                    pltpu.sync_copy(dst2d_h.at[pl.ds(base, CH)], dsti)
                    for b in range(NB):
                        pltpu.async_copy(tbl.at[srci.at[b]], rows[b], sems[b])
                    def steady(j2, carry2):
                        for b in range(NB):
                            jj = j2 * NB + b
                            pltpu.make_async_copy(
                                tbl.at[srci.at[jj]], rows[b], sems[b]).wait()
                            pltpu.sync_copy(rows[b], acc.at[dsti.at[jj]], add=True)
                            pltpu.async_copy(tbl.at[srci.at[jj + NB]], rows[b],
                                             sems[b])
                        return carry2
                    lax.fori_loop(0, CH // NB - 1, steady, 0)
                    for b in range(NB):
                        jj = CH - NB + b
                        pltpu.make_async_copy(
                            tbl.at[srci.at[jj]], rows[b], sems[b]).wait()
                        pltpu.sync_copy(rows[b], acc.at[dsti.at[jj]], add=True)
                    return carry
                lax.fori_loop(0, nbatch // CH, chunk, 0)

            if per_core_tbl:
                @pl.when(c == 0)
                def _():
                    run(tables[0])
                @pl.when(c == 1)
                def _():
                    run(tables[1])
            else:
                run(tables[0])

        plsc.subcore_barrier()

        @pl.when(c == 0)
        def _():
            pltpu.sync_copy(acc.at[pl.ds(s * 640, 640)], out0.at[pl.ds(s * 640, 640)])
        @pl.when(c == 1)
        def _():
            pltpu.sync_copy(acc.at[pl.ds(s * 640, 640)], out1.at[pl.ds(s * 640, 640)])

    return prop


def _make_sc_prop64(two_pass):
    n_t = 4 if two_pass else 2
    nbatch = EPAD // 16 // G
    CH = 32
    out_sd = jax.ShapeDtypeStruct((NPAD, 64), F32)
    scratch = [
        pltpu.VMEM_SHARED((NPAD, 64), F32),
        pltpu.VMEM_SHARED((NPAD, 64), F32),
        pltpu.VMEM((32, 64), F32),
        pltpu.VMEM((CH, G), jnp.int32),
        pltpu.VMEM((CH, G), jnp.int32),
    ]
    scratch += [pltpu.VMEM((G, 64), F32) for _ in range(NB)]
    scratch += [pltpu.SemaphoreType.DMA for _ in range(NB)]

    @functools.partial(
        pl.kernel,
        out_type=tuple(out_sd for _ in range(n_t)),
        scratch_types=scratch,
        mesh=_MESH,
    )
    def prop(*refs):
        tables = refs[:n_t]
        src2d_h, dst2d_h = refs[n_t], refs[n_t + 1]
        outs = refs[n_t + 2:2 * n_t + 2]
        k = 2 * n_t + 2
        tbl_sh, acc, zbuf, srci, dsti = refs[k:k + 5]; k += 5
        rows = refs[k:k + NB]; k += NB
        sems = refs[k:k + NB]

        c = lax.axis_index("c")
        s = lax.axis_index("s")

        def do_pass(tbl_h, out_h):
            def zb(i, carry):
                zbuf[i // 4, pl.ds((i % 4) * 16, 16)] = jnp.zeros((16,), F32)
                return carry
            lax.fori_loop(0, 128, zb, 0)
            def zcp(kk, carry):
                pltpu.sync_copy(zbuf, acc.at[pl.ds(s * 640 + kk * 32, 32)])
                return carry
            lax.fori_loop(0, 20, zcp, 0)
            pltpu.sync_copy(tbl_h.at[pl.ds(s * 640, 640)],
                            tbl_sh.at[pl.ds(s * 640, 640)])
            plsc.subcore_barrier()

            def chunk(ci, carry):
                base = s * nbatch + ci * CH
                pltpu.sync_copy(src2d_h.at[pl.ds(base, CH)], srci)
                pltpu.sync_copy(dst2d_h.at[pl.ds(base, CH)], dsti)
                for b in range(NB):
                    pltpu.async_copy(tbl_sh.at[srci.at[b]], rows[b], sems[b])
                def steady(j2, carry2):
                    for b in range(NB):
                        jj = j2 * NB + b
                        pltpu.make_async_copy(
                            tbl_sh.at[srci.at[jj]], rows[b], sems[b]).wait()
                        pltpu.sync_copy(rows[b], acc.at[dsti.at[jj]], add=True)
                        pltpu.async_copy(tbl_sh.at[srci.at[jj + NB]], rows[b],
                                         sems[b])
                    return carry2
                lax.fori_loop(0, CH // NB - 1, steady, 0)
                for b in range(NB):
                    jj = CH - NB + b
                    pltpu.make_async_copy(
                        tbl_sh.at[srci.at[jj]], rows[b], sems[b]).wait()
                    pltpu.sync_copy(rows[b], acc.at[dsti.at[jj]], add=True)
                return carry
            lax.fori_loop(0, nbatch // CH, chunk, 0)
            plsc.subcore_barrier()
            pltpu.sync_copy(acc.at[pl.ds(s * 640, 640)],
                            out_h.at[pl.ds(s * 640, 640)])

        if two_pass:
            @pl.when(c == 0)
            def _():
                do_pass(tables[0], outs[0])
                do_pass(tables[1], outs[1])
            @pl.when(c == 1)
            def _():
                do_pass(tables[2], outs[2])
                do_pass(tables[3], outs[3])
        else:
            @pl.when(c == 0)
            def _():
                do_pass(tables[0], outs[0])
            @pl.when(c == 1)
            def _():
                do_pass(tables[1], outs[1])

    return prop


_sc_deg = _make_sc_prop(1, const_ones=True)
_sc_prop1 = _make_sc_prop(1)
_sc_prop128 = _make_sc_prop64(two_pass=False)
_sc_prop256 = _make_sc_prop64(two_pass=True)


def _tca_body(d0, d1, x, dis_o, t0a_o, t0b_o):
    deg = d0[...] + d1[...] + 1.0
    dis = lax.rsqrt(deg)
    dis_o[...] = dis
    t0 = x[...] * dis
    t0a_o[...] = t0[:, :64]
    t0b_o[...] = t0[:, 64:]


def _tc_a(d0, d1, x):
    R = 2000
    return pl.pallas_call(
        _tca_body,
        grid=(N // R,),
        in_specs=[
            pl.BlockSpec((R, 1), lambda i: (i, 0)),
            pl.BlockSpec((R, 1), lambda i: (i, 0)),
            pl.BlockSpec((R, 128), lambda i: (i, 0)),
        ],
        out_specs=[
            pl.BlockSpec((R, 1), lambda i: (i, 0)),
            pl.BlockSpec((R, 64), lambda i: (i, 0)),
            pl.BlockSpec((R, 64), lambda i: (i, 0)),
        ],
        out_shape=[
            jax.ShapeDtypeStruct((N, 1), F32),
            jax.ShapeDtypeStruct((NPAD, 64), F32),
            jax.ShapeDtypeStruct((NPAD, 64), F32),
        ],
    )(d0, d1, x)


def _tcb_body(a0, a1, t0a, t0b, dis, W1, b1, o0, o1, o2, o3):
    d = dis[...]
    z = jnp.concatenate(
        [(a0[...] + t0a[...]) * d, (a1[...] + t0b[...]) * d], axis=1)
    h = jnp.maximum(jnp.dot(z, W1[...], preferred_element_type=F32) + b1[...], 0.0)
    t1 = h * d
    o0[...] = t1[:, 0:64]
    o1[...] = t1[:, 64:128]
    o2[...] = t1[:, 128:192]
    o3[...] = t1[:, 192:256]


def _tc_b(a0, a1, t0a, t0b, dis, W1, b1):
    R = 1000
    q = jax.ShapeDtypeStruct((NPAD, 64), F32)
    return pl.pallas_call(
        _tcb_body,
        grid=(N // R,),
        in_specs=[
            pl.BlockSpec((R, 64), lambda i: (i, 0)),
            pl.BlockSpec((R, 64), lambda i: (i, 0)),
            pl.BlockSpec((R, 64), lambda i: (i, 0)),
            pl.BlockSpec((R, 64), lambda i: (i, 0)),
            pl.BlockSpec((R, 1), lambda i: (i, 0)),
            pl.BlockSpec((128, 256), lambda i: (0, 0)),
            pl.BlockSpec((1, 256), lambda i: (0, 0)),
        ],
        out_specs=[pl.BlockSpec((R, 64), lambda i: (i, 0)) for _ in range(4)],
        out_shape=[q, q, q, q],
    )(a0, a1, t0a, t0b, dis, W1, b1)


def _tcc_body(c0, c1, c2, c3, t0, t1, t2, t3r, dis, W2, b2, W3, o):
    d = dis[...]
    z = jnp.concatenate(
        [(c0[...] + t0[...]) * d, (c1[...] + t1[...]) * d,
         (c2[...] + t2[...]) * d, (c3[...] + t3r[...]) * d], axis=1)
    h = jnp.maximum(jnp.dot(z, W2[...], preferred_element_type=F32) + b2[...], 0.0)
    o[...] = jnp.dot(h, W3[...], preferred_element_type=F32) * d


def _tc_c(cq, tq, dis, W2, b2, W3):
    R = 1000
    return pl.pallas_call(
        _tcc_body,
        grid=(N // R,),
        in_specs=(
            [pl.BlockSpec((R, 64), lambda i: (i, 0)) for _ in range(8)]
            + [
                pl.BlockSpec((R, 1), lambda i: (i, 0)),
                pl.BlockSpec((256, 512), lambda i: (0, 0)),
                pl.BlockSpec((1, 512), lambda i: (0, 0)),
                pl.BlockSpec((512, 1), lambda i: (0, 0)),
            ]
        ),
        out_specs=pl.BlockSpec((R, 1), lambda i: (i, 0)),
        out_shape=jax.ShapeDtypeStruct((N, 1), F32),
    )(*cq, *tq, dis, W2, b2, W3)


def _tcd_body(p0, p1, t3, dis, b3, o):
    z = (p0[...] + p1[...] + t3[...]) * dis[...] + b3[...]
    o[...] = 1.0 / (1.0 + jnp.exp(-z))


def _tc_d(p0, p1, t3, dis, b3):
    R = 2000
    return pl.pallas_call(
        _tcd_body,
        grid=(N // R,),
        in_specs=[
            pl.BlockSpec((R, 1), lambda i: (i, 0)),
            pl.BlockSpec((R, 1), lambda i: (i, 0)),
            pl.BlockSpec((R, 1), lambda i: (i, 0)),
            pl.BlockSpec((R, 1), lambda i: (i, 0)),
            pl.BlockSpec((1, 1), lambda i: (0, 0)),
        ],
        out_specs=pl.BlockSpec((R, 1), lambda i: (i, 0)),
        out_shape=jax.ShapeDtypeStruct((N, 1), F32),
    )(p0, p1, t3, dis, b3)


def kernel(x, edge_index, W1, b1, W2, b2, W3, b3):
    src = edge_index[0].astype(jnp.int32)
    dst = edge_index[1].astype(jnp.int32)
    npadrows = NPAD - N
    pad_dst = N + (jnp.arange(EPAD - E, dtype=jnp.int32) % npadrows)
    src2d = jnp.concatenate(
        [src, jnp.zeros((EPAD - E,), jnp.int32)]).reshape(EPAD // G, G)
    dst2d = jnp.concatenate([dst, pad_dst]).reshape(EPAD // G, G)

    d0, d1 = _sc_deg(dst2d)
    dis, t0a, t0b = _tc_a(d0.reshape(NPAD, 1), d1.reshape(NPAD, 1), x)
    a0, a1 = _sc_prop128(t0a, t0b, src2d, dst2d)
    t1q = _tc_b(a0, a1, t0a, t0b, dis, W1, b1.reshape(1, -1))
    cq = _sc_prop256(*t1q, src2d, dst2d)
    t3 = _tc_c(cq, t1q, dis, W2, b2.reshape(1, -1), W3)
    p0, p1 = _sc_prop1(t3.reshape(N), src2d, dst2d)
    return _tc_d(p0.reshape(NPAD, 1), p1.reshape(NPAD, 1), t3, dis,
                 b3.reshape(1, 1))


# speedup vs baseline: 1.9205x; 1.0824x over previous
import functools

import jax
import jax.numpy as jnp
from jax import lax
from jax.experimental import pallas as pl
from jax.experimental.pallas import tpu as pltpu
from jax.experimental.pallas import tpu_sc as plsc

N = 10000
E = 320000
NPAD = 10240
EPAD = 32 * NPAD
G = 128
NB = 4
NBW = 2
F32 = jnp.float32

_MESH = plsc.VectorSubcoreMesh(core_axis_name="c", subcore_axis_name="s")


def _make_sc_prop(feat, feat_split=False, const_ones=False, two_tables=False):
    nbatch = (EPAD // 16 if feat_split else EPAD // 32) // G
    CH = 32 if feat_split else 40
    wide = feat > 1
    acc_shape = (NPAD, feat) if wide else (NPAD,)
    row_shape = (G, feat) if wide else (G,)
    zrows = 32
    out_sd = jax.ShapeDtypeStruct(acc_shape, F32)

    per_core_tbl = feat_split or two_tables
    n_tables = 0 if const_ones else (2 if per_core_tbl else 1)
    stage_tbl = (not const_ones) and not wide
    scratch = [pltpu.VMEM_SHARED(acc_shape, F32)]
    if stage_tbl:
        scratch.append(pltpu.VMEM_SHARED((NPAD,), F32))
    scratch.append(pltpu.VMEM((zrows, feat) if wide else (640,), F32))
    if not const_ones:
        scratch.append(pltpu.VMEM((CH, G), jnp.int32))
    scratch.append(pltpu.VMEM((CH, G), jnp.int32))
    scratch += [pltpu.VMEM(row_shape, F32) for _ in range(NB)]
    scratch += [pltpu.SemaphoreType.DMA for _ in range(2 * NB)]

    @functools.partial(
        pl.kernel,
        out_type=(out_sd, out_sd),
        scratch_types=scratch,
        mesh=_MESH,
    )
    def prop(*refs):
        k = n_tables
        tables = refs[:k]
        if not const_ones:
            src2d_h = refs[k]; k += 1
        dst2d_h = refs[k]; k += 1
        out0, out1 = refs[k], refs[k + 1]; k += 2
        acc = refs[k]; k += 1
        if stage_tbl:
            tbl_sh = refs[k]; k += 1
        zbuf = refs[k]; k += 1
        if not const_ones:
            srci = refs[k]; k += 1
        dsti = refs[k]; k += 1
        rows = refs[k:k + NB]; k += NB
        sems = refs[k:k + NB]; k += NB
        ssems = refs[k:k + NB]

        c = lax.axis_index("c")
        s = lax.axis_index("s")

        if wide:
            def zb(i, carry):
                zbuf[i // 8, pl.ds((i % 8) * 16, 16)] = jnp.zeros((16,), F32)
                return carry
            lax.fori_loop(0, zrows * 8, zb, 0)
            def zcp(kk, carry):
                pltpu.sync_copy(zbuf, acc.at[pl.ds(s * 640 + kk * zrows, zrows)])
                return carry
            lax.fori_loop(0, 640 // zrows, zcp, 0)
        else:
            def zb(i, carry):
                zbuf[pl.ds(i * 16, 16)] = jnp.zeros((16,), F32)
                return carry
            lax.fori_loop(0, 40, zb, 0)
            pltpu.sync_copy(zbuf, acc.at[pl.ds(s * 640, 640)])
        if stage_tbl:
            pltpu.sync_copy(tables[0].at[pl.ds(s * 640, 640)],
                            tbl_sh.at[pl.ds(s * 640, 640)])
        plsc.subcore_barrier()

        ibase = s * nbatch if feat_split else c * 16 * nbatch + s * nbatch

        if const_ones:
            for b in range(NB):
                def ob(i, carry, _b=b):
                    rows[_b][pl.ds(i * 16, 16)] = jnp.ones((16,), F32)
                    return carry
                lax.fori_loop(0, G // 16, ob, 0)

            def chunk(ci, carry):
                pltpu.sync_copy(dst2d_h.at[pl.ds(ibase + ci * CH, CH)], dsti)
                for b in range(NB):
                    pltpu.async_copy(rows[b], acc.at[dsti.at[b]], sems[b], add=True)
                def steady(j2, carry2):
                    for b in range(NB):
                        jj = j2 * NB + b
                        pltpu.make_async_copy(
                            rows[b], acc.at[dsti.at[jj - NB]], sems[b]).wait()
                        pltpu.async_copy(rows[b], acc.at[dsti.at[jj]], sems[b],
                                         add=True)
                    return carry2
                lax.fori_loop(1, CH // NB, steady, 0)
                for b in range(NB):
                    jj = CH - NB + b
                    pltpu.make_async_copy(rows[b], acc.at[dsti.at[jj]], sems[b]).wait()
                return carry
            lax.fori_loop(0, nbatch // CH, chunk, 0)
        else:
            def run(tbl):
                def chunk(ci, carry):
                    base = ibase + ci * CH
                    pltpu.sync_copy(src2d_h.at[pl.ds(base, CH)], srci)
                    pltpu.sync_copy(dst2d_h.at[pl.ds(base, CH)], dsti)
                    for b in range(NB):
                        pltpu.async_copy(tbl.at[srci.at[b]], rows[b], sems[b])
                    def steady(j2, carry2):
                        for b in range(NB):
                            jj = j2 * NB + b
                            pltpu.make_async_copy(
                                tbl.at[srci.at[jj]], rows[b], sems[b]).wait()
                            pltpu.sync_copy(rows[b], acc.at[dsti.at[jj]], add=True)
                            pltpu.async_copy(tbl.at[srci.at[jj + NB]], rows[b],
                                             sems[b])
                        return carry2
                    lax.fori_loop(0, CH // NB - 1, steady, 0)
                    for b in range(NB):
                        jj = CH - NB + b
                        pltpu.make_async_copy(
                            tbl.at[srci.at[jj]], rows[b], sems[b]).wait()
                        pltpu.sync_copy(rows[b], acc.at[dsti.at[jj]], add=True)
                    return carry
                lax.fori_loop(0, nbatch // CH, chunk, 0)

            if per_core_tbl:
                @pl.when(c == 0)
                def _():
                    run(tables[0])
                @pl.when(c == 1)
                def _():
                    run(tables[1])
            elif stage_tbl:
                run(tbl_sh)
            else:
                run(tables[0])

        plsc.subcore_barrier()

        @pl.when(c == 0)
        def _():
            pltpu.sync_copy(acc.at[pl.ds(s * 640, 640)], out0.at[pl.ds(s * 640, 640)])
        @pl.when(c == 1)
        def _():
            pltpu.sync_copy(acc.at[pl.ds(s * 640, 640)], out1.at[pl.ds(s * 640, 640)])

    return prop


def _make_sc_prop64(two_pass):
    n_t = 4 if two_pass else 2
    nbatch = EPAD // 16 // G
    CH = 32
    out_sd = jax.ShapeDtypeStruct((NPAD, 64), F32)
    scratch = [
        pltpu.VMEM_SHARED((NPAD, 64), F32),
        pltpu.VMEM_SHARED((NPAD, 64), F32),
        pltpu.VMEM((32, 64), F32),
        pltpu.VMEM((CH, G), jnp.int32),
        pltpu.VMEM((CH, G), jnp.int32),
    ]
    scratch += [pltpu.VMEM((G, 64), F32) for _ in range(NBW)]
    scratch += [pltpu.SemaphoreType.DMA for _ in range(NBW)]

    @functools.partial(
        pl.kernel,
        out_type=tuple(out_sd for _ in range(n_t)),
        scratch_types=scratch,
        mesh=_MESH,
    )
    def prop(*refs):
        tables = refs[:n_t]
        src2d_h, dst2d_h = refs[n_t], refs[n_t + 1]
        outs = refs[n_t + 2:2 * n_t + 2]
        k = 2 * n_t + 2
        tbl_sh, acc, zbuf, srci, dsti = refs[k:k + 5]; k += 5
        rows = refs[k:k + NBW]; k += NBW
        sems = refs[k:k + NBW]

        c = lax.axis_index("c")
        s = lax.axis_index("s")

        def do_pass(tbl_h, out_h):
            def zb(i, carry):
                zbuf[i // 4, pl.ds((i % 4) * 16, 16)] = jnp.zeros((16,), F32)
                return carry
            lax.fori_loop(0, 128, zb, 0)
            def zcp(kk, carry):
                pltpu.sync_copy(zbuf, acc.at[pl.ds(s * 640 + kk * 32, 32)])
                return carry
            lax.fori_loop(0, 20, zcp, 0)
            pltpu.sync_copy(tbl_h.at[pl.ds(s * 640, 640)],
                            tbl_sh.at[pl.ds(s * 640, 640)])
            plsc.subcore_barrier()

            def chunk(ci, carry):
                base = s * nbatch + ci * CH
                pltpu.sync_copy(src2d_h.at[pl.ds(base, CH)], srci)
                pltpu.sync_copy(dst2d_h.at[pl.ds(base, CH)], dsti)
                for b in range(NBW):
                    pltpu.async_copy(tbl_sh.at[srci.at[b]], rows[b], sems[b])
                def steady(j2, carry2):
                    for b in range(NBW):
                        jj = j2 * NBW + b
                        pltpu.make_async_copy(
                            tbl_sh.at[srci.at[jj]], rows[b], sems[b]).wait()
                        pltpu.sync_copy(rows[b], acc.at[dsti.at[jj]], add=True)
                        pltpu.async_copy(tbl_sh.at[srci.at[jj + NBW]], rows[b],
                                         sems[b])
                    return carry2
                lax.fori_loop(0, CH // NBW - 1, steady, 0)
                for b in range(NBW):
                    jj = CH - NBW + b
                    pltpu.make_async_copy(
                        tbl_sh.at[srci.at[jj]], rows[b], sems[b]).wait()
                    pltpu.sync_copy(rows[b], acc.at[dsti.at[jj]], add=True)
                return carry
            lax.fori_loop(0, nbatch // CH, chunk, 0)
            plsc.subcore_barrier()
            pltpu.sync_copy(acc.at[pl.ds(s * 640, 640)],
                            out_h.at[pl.ds(s * 640, 640)])

        if two_pass:
            @pl.when(c == 0)
            def _():
                do_pass(tables[0], outs[0])
                do_pass(tables[1], outs[1])
            @pl.when(c == 1)
            def _():
                do_pass(tables[2], outs[2])
                do_pass(tables[3], outs[3])
        else:
            @pl.when(c == 0)
            def _():
                do_pass(tables[0], outs[0])
            @pl.when(c == 1)
            def _():
                do_pass(tables[1], outs[1])

    return prop


_sc_deg = _make_sc_prop(1, const_ones=True)
_sc_prop1 = _make_sc_prop(1)
_sc_prop128 = _make_sc_prop64(two_pass=False)
_sc_prop256 = _make_sc_prop64(two_pass=True)


def _tca_body(d0, d1, x, dis_o, t0a_o, t0b_o):
    deg = d0[...] + d1[...] + 1.0
    dis = lax.rsqrt(deg)
    dis_o[...] = dis
    t0 = x[...] * dis
    t0a_o[...] = t0[:, :64]
    t0b_o[...] = t0[:, 64:]


def _tc_a(d0, d1, x):
    R = 2000
    return pl.pallas_call(
        _tca_body,
        grid=(N // R,),
        in_specs=[
            pl.BlockSpec((R, 1), lambda i: (i, 0)),
            pl.BlockSpec((R, 1), lambda i: (i, 0)),
            pl.BlockSpec((R, 128), lambda i: (i, 0)),
        ],
        out_specs=[
            pl.BlockSpec((R, 1), lambda i: (i, 0)),
            pl.BlockSpec((R, 64), lambda i: (i, 0)),
            pl.BlockSpec((R, 64), lambda i: (i, 0)),
        ],
        out_shape=[
            jax.ShapeDtypeStruct((N, 1), F32),
            jax.ShapeDtypeStruct((NPAD, 64), F32),
            jax.ShapeDtypeStruct((NPAD, 64), F32),
        ],
    )(d0, d1, x)


def _tcb_body(a0, a1, t0a, t0b, dis, W1, b1, o0, o1, o2, o3):
    d = dis[...]
    z = jnp.concatenate(
        [(a0[...] + t0a[...]) * d, (a1[...] + t0b[...]) * d], axis=1)
    h = jnp.maximum(jnp.dot(z, W1[...], preferred_element_type=F32) + b1[...], 0.0)
    t1 = h * d
    o0[...] = t1[:, 0:64]
    o1[...] = t1[:, 64:128]
    o2[...] = t1[:, 128:192]
    o3[...] = t1[:, 192:256]


def _tc_b(a0, a1, t0a, t0b, dis, W1, b1):
    R = 1000
    q = jax.ShapeDtypeStruct((NPAD, 64), F32)
    return pl.pallas_call(
        _tcb_body,
        grid=(N // R,),
        in_specs=[
            pl.BlockSpec((R, 64), lambda i: (i, 0)),
            pl.BlockSpec((R, 64), lambda i: (i, 0)),
            pl.BlockSpec((R, 64), lambda i: (i, 0)),
            pl.BlockSpec((R, 64), lambda i: (i, 0)),
            pl.BlockSpec((R, 1), lambda i: (i, 0)),
            pl.BlockSpec((128, 256), lambda i: (0, 0)),
            pl.BlockSpec((1, 256), lambda i: (0, 0)),
        ],
        out_specs=[pl.BlockSpec((R, 64), lambda i: (i, 0)) for _ in range(4)],
        out_shape=[q, q, q, q],
    )(a0, a1, t0a, t0b, dis, W1, b1)


def _tcc_body(c0, c1, c2, c3, t0, t1, t2, t3r, dis, W2, b2, W3, o):
    d = dis[...]
    z = jnp.concatenate(
        [(c0[...] + t0[...]) * d, (c1[...] + t1[...]) * d,
         (c2[...] + t2[...]) * d, (c3[...] + t3r[...]) * d], axis=1)
    h = jnp.maximum(jnp.dot(z, W2[...], preferred_element_type=F32) + b2[...], 0.0)
    o[...] = jnp.dot(h, W3[...], preferred_element_type=F32) * d


def _tc_c(cq, tq, dis, W2, b2, W3):
    R = 1000
    return pl.pallas_call(
        _tcc_body,
        grid=(N // R,),
        in_specs=(
            [pl.BlockSpec((R, 64), lambda i: (i, 0)) for _ in range(8)]
            + [
                pl.BlockSpec((R, 1), lambda i: (i, 0)),
                pl.BlockSpec((256, 512), lambda i: (0, 0)),
                pl.BlockSpec((1, 512), lambda i: (0, 0)),
                pl.BlockSpec((512, 1), lambda i: (0, 0)),
            ]
        ),
        out_specs=pl.BlockSpec((R, 1), lambda i: (i, 0)),
        out_shape=jax.ShapeDtypeStruct((NPAD, 1), F32),
    )(*cq, *tq, dis, W2, b2, W3)


def _tcd_body(p0, p1, t3, dis, b3, o):
    z = (p0[...] + p1[...] + t3[...]) * dis[...] + b3[...]
    o[...] = 1.0 / (1.0 + jnp.exp(-z))


def _tc_d(p0, p1, t3, dis, b3):
    R = 2000
    return pl.pallas_call(
        _tcd_body,
        grid=(N // R,),
        in_specs=[
            pl.BlockSpec((R, 1), lambda i: (i, 0)),
            pl.BlockSpec((R, 1), lambda i: (i, 0)),
            pl.BlockSpec((R, 1), lambda i: (i, 0)),
            pl.BlockSpec((R, 1), lambda i: (i, 0)),
            pl.BlockSpec((1, 1), lambda i: (0, 0)),
        ],
        out_specs=pl.BlockSpec((R, 1), lambda i: (i, 0)),
        out_shape=jax.ShapeDtypeStruct((N, 1), F32),
    )(p0, p1, t3, dis, b3)


def kernel(x, edge_index, W1, b1, W2, b2, W3, b3):
    src = edge_index[0].astype(jnp.int32)
    dst = edge_index[1].astype(jnp.int32)
    npadrows = NPAD - N
    pad_dst = N + (jnp.arange(EPAD - E, dtype=jnp.int32) % npadrows)
    src2d = jnp.concatenate(
        [src, jnp.zeros((EPAD - E,), jnp.int32)]).reshape(EPAD // G, G)
    dst2d = jnp.concatenate([dst, pad_dst]).reshape(EPAD // G, G)

    d0, d1 = _sc_deg(dst2d)
    dis, t0a, t0b = _tc_a(d0.reshape(NPAD, 1), d1.reshape(NPAD, 1), x)
    a0, a1 = _sc_prop128(t0a, t0b, src2d, dst2d)
    t1q = _tc_b(a0, a1, t0a, t0b, dis, W1, b1.reshape(1, -1))
    cq = _sc_prop256(*t1q, src2d, dst2d)
    t3 = _tc_c(cq, t1q, dis, W2, b2.reshape(1, -1), W3)
    p0, p1 = _sc_prop1(t3.reshape(NPAD), src2d, dst2d)
    return _tc_d(p0.reshape(NPAD, 1), p1.reshape(NPAD, 1), t3, dis,
                 b3.reshape(1, 1))


# speedup vs baseline: 1.9410x; 1.0107x over previous
import functools

import jax
import jax.numpy as jnp
from jax import lax
from jax.experimental import pallas as pl
from jax.experimental.pallas import tpu as pltpu
from jax.experimental.pallas import tpu_sc as plsc

N = 10000
E = 320000
NPAD = 10240
EPAD = 32 * NPAD
G = 128
NB = 4
NBW = 4
F32 = jnp.float32

_MESH = plsc.VectorSubcoreMesh(core_axis_name="c", subcore_axis_name="s")


def _make_sc_prop(feat, feat_split=False, const_ones=False, two_tables=False):
    nbatch = (EPAD // 16 if feat_split else EPAD // 32) // G
    CH = 32 if feat_split else 40
    wide = feat > 1
    acc_shape = (NPAD, feat) if wide else (NPAD,)
    row_shape = (G, feat) if wide else (G,)
    zrows = 32
    out_sd = jax.ShapeDtypeStruct(acc_shape, F32)

    per_core_tbl = feat_split or two_tables
    n_tables = 0 if const_ones else (2 if per_core_tbl else 1)
    stage_tbl = (not const_ones) and not wide
    scratch = [pltpu.VMEM_SHARED(acc_shape, F32)]
    if stage_tbl:
        scratch.append(pltpu.VMEM_SHARED((NPAD,), F32))
    scratch.append(pltpu.VMEM((zrows, feat) if wide else (640,), F32))
    if not const_ones:
        scratch.append(pltpu.VMEM((CH, G), jnp.int32))
    scratch.append(pltpu.VMEM((CH, G), jnp.int32))
    scratch += [pltpu.VMEM(row_shape, F32) for _ in range(NB)]
    scratch += [pltpu.SemaphoreType.DMA for _ in range(2 * NB)]

    @functools.partial(
        pl.kernel,
        out_type=(out_sd, out_sd),
        scratch_types=scratch,
        mesh=_MESH,
    )
    def prop(*refs):
        k = n_tables
        tables = refs[:k]
        if not const_ones:
            src2d_h = refs[k]; k += 1
        dst2d_h = refs[k]; k += 1
        out0, out1 = refs[k], refs[k + 1]; k += 2
        acc = refs[k]; k += 1
        if stage_tbl:
            tbl_sh = refs[k]; k += 1
        zbuf = refs[k]; k += 1
        if not const_ones:
            srci = refs[k]; k += 1
        dsti = refs[k]; k += 1
        rows = refs[k:k + NB]; k += NB
        sems = refs[k:k + NB]; k += NB
        ssems = refs[k:k + NB]

        c = lax.axis_index("c")
        s = lax.axis_index("s")

        if wide:
            def zb(i, carry):
                zbuf[i // 8, pl.ds((i % 8) * 16, 16)] = jnp.zeros((16,), F32)
                return carry
            lax.fori_loop(0, zrows * 8, zb, 0)
            def zcp(kk, carry):
                pltpu.sync_copy(zbuf, acc.at[pl.ds(s * 640 + kk * zrows, zrows)])
                return carry
            lax.fori_loop(0, 640 // zrows, zcp, 0)
        else:
            def zb(i, carry):
                zbuf[pl.ds(i * 16, 16)] = jnp.zeros((16,), F32)
                return carry
            lax.fori_loop(0, 40, zb, 0)
            pltpu.sync_copy(zbuf, acc.at[pl.ds(s * 640, 640)])
        if stage_tbl:
            pltpu.sync_copy(tables[0].at[pl.ds(s * 640, 640)],
                            tbl_sh.at[pl.ds(s * 640, 640)])
        plsc.subcore_barrier()

        ibase = s * nbatch if feat_split else c * 16 * nbatch + s * nbatch

        if const_ones:
            for b in range(NB):
                def ob(i, carry, _b=b):
                    rows[_b][pl.ds(i * 16, 16)] = jnp.ones((16,), F32)
                    return carry
                lax.fori_loop(0, G // 16, ob, 0)

            def chunk(ci, carry):
                pltpu.sync_copy(dst2d_h.at[pl.ds(ibase + ci * CH, CH)], dsti)
                for b in range(NB):
                    pltpu.async_copy(rows[b], acc.at[dsti.at[b]], sems[b], add=True)
                def steady(j2, carry2):
                    for b in range(NB):
                        jj = j2 * NB + b
                        pltpu.make_async_copy(
                            rows[b], acc.at[dsti.at[jj - NB]], sems[b]).wait()
                        pltpu.async_copy(rows[b], acc.at[dsti.at[jj]], sems[b],
                                         add=True)
                    return carry2
                lax.fori_loop(1, CH // NB, steady, 0)
                for b in range(NB):
                    jj = CH - NB + b
                    pltpu.make_async_copy(rows[b], acc.at[dsti.at[jj]], sems[b]).wait()
                return carry
            lax.fori_loop(0, nbatch // CH, chunk, 0)
        else:
            def run(tbl):
                def chunk(ci, carry):
                    base = ibase + ci * CH
                    pltpu.sync_copy(src2d_h.at[pl.ds(base, CH)], srci)
                    pltpu.sync_copy(dst2d_h.at[pl.ds(base, CH)], dsti)
                    for b in range(NB):
                        pltpu.async_copy(tbl.at[srci.at[b]], rows[b], sems[b])
                    def steady(j2, carry2):
                        for b in range(NB):
                            jj = j2 * NB + b
                            pltpu.make_async_copy(
                                tbl.at[srci.at[jj]], rows[b], sems[b]).wait()
                            pltpu.sync_copy(rows[b], acc.at[dsti.at[jj]], add=True)
                            pltpu.async_copy(tbl.at[srci.at[jj + NB]], rows[b],
                                             sems[b])
                        return carry2
                    lax.fori_loop(0, CH // NB - 1, steady, 0)
                    for b in range(NB):
                        jj = CH - NB + b
                        pltpu.make_async_copy(
                            tbl.at[srci.at[jj]], rows[b], sems[b]).wait()
                        pltpu.sync_copy(rows[b], acc.at[dsti.at[jj]], add=True)
                    return carry
                lax.fori_loop(0, nbatch // CH, chunk, 0)

            if per_core_tbl:
                @pl.when(c == 0)
                def _():
                    run(tables[0])
                @pl.when(c == 1)
                def _():
                    run(tables[1])
            elif stage_tbl:
                run(tbl_sh)
            else:
                run(tables[0])

        plsc.subcore_barrier()

        @pl.when(c == 0)
        def _():
            pltpu.sync_copy(acc.at[pl.ds(s * 640, 640)], out0.at[pl.ds(s * 640, 640)])
        @pl.when(c == 1)
        def _():
            pltpu.sync_copy(acc.at[pl.ds(s * 640, 640)], out1.at[pl.ds(s * 640, 640)])

    return prop


def _make_sc_prop64(two_pass):
    n_t = 4 if two_pass else 2
    G2 = 64
    nbatch = EPAD // 16 // G2
    CH = 32
    out_sd = jax.ShapeDtypeStruct((NPAD, 64), F32)
    scratch = [
        pltpu.VMEM_SHARED((NPAD, 64), F32),
        pltpu.VMEM_SHARED((NPAD, 64), F32),
        pltpu.VMEM((32, 64), F32),
        pltpu.VMEM((CH, G2), jnp.int32),
        pltpu.VMEM((CH, G2), jnp.int32),
    ]
    scratch += [pltpu.VMEM((G2, 64), F32) for _ in range(NBW)]
    scratch += [pltpu.SemaphoreType.DMA for _ in range(NBW)]

    @functools.partial(
        pl.kernel,
        out_type=tuple(out_sd for _ in range(n_t)),
        scratch_types=scratch,
        mesh=_MESH,
    )
    def prop(*refs):
        tables = refs[:n_t]
        src2d_h, dst2d_h = refs[n_t], refs[n_t + 1]
        outs = refs[n_t + 2:2 * n_t + 2]
        k = 2 * n_t + 2
        tbl_sh, acc, zbuf, srci, dsti = refs[k:k + 5]; k += 5
        rows = refs[k:k + NBW]; k += NBW
        sems = refs[k:k + NBW]

        c = lax.axis_index("c")
        s = lax.axis_index("s")

        def do_pass(tbl_h, out_h):
            def zb(i, carry):
                zbuf[i // 4, pl.ds((i % 4) * 16, 16)] = jnp.zeros((16,), F32)
                return carry
            lax.fori_loop(0, 128, zb, 0)
            def zcp(kk, carry):
                pltpu.sync_copy(zbuf, acc.at[pl.ds(s * 640 + kk * 32, 32)])
                return carry
            lax.fori_loop(0, 20, zcp, 0)
            pltpu.sync_copy(tbl_h.at[pl.ds(s * 640, 640)],
                            tbl_sh.at[pl.ds(s * 640, 640)])
            plsc.subcore_barrier()

            def chunk(ci, carry):
                base = s * nbatch + ci * CH
                pltpu.sync_copy(src2d_h.at[pl.ds(base, CH)], srci)
                pltpu.sync_copy(dst2d_h.at[pl.ds(base, CH)], dsti)
                for b in range(NBW):
                    pltpu.async_copy(tbl_sh.at[srci.at[b]], rows[b], sems[b])
                def steady(j2, carry2):
                    for b in range(NBW):
                        jj = j2 * NBW + b
                        pltpu.make_async_copy(
                            tbl_sh.at[srci.at[jj]], rows[b], sems[b]).wait()
                        pltpu.sync_copy(rows[b], acc.at[dsti.at[jj]], add=True)
                        pltpu.async_copy(tbl_sh.at[srci.at[jj + NBW]], rows[b],
                                         sems[b])
                    return carry2
                lax.fori_loop(0, CH // NBW - 1, steady, 0)
                for b in range(NBW):
                    jj = CH - NBW + b
                    pltpu.make_async_copy(
                        tbl_sh.at[srci.at[jj]], rows[b], sems[b]).wait()
                    pltpu.sync_copy(rows[b], acc.at[dsti.at[jj]], add=True)
                return carry
            lax.fori_loop(0, nbatch // CH, chunk, 0)
            plsc.subcore_barrier()
            pltpu.sync_copy(acc.at[pl.ds(s * 640, 640)],
                            out_h.at[pl.ds(s * 640, 640)])

        if two_pass:
            @pl.when(c == 0)
            def _():
                do_pass(tables[0], outs[0])
                do_pass(tables[1], outs[1])
            @pl.when(c == 1)
            def _():
                do_pass(tables[2], outs[2])
                do_pass(tables[3], outs[3])
        else:
            @pl.when(c == 0)
            def _():
                do_pass(tables[0], outs[0])
            @pl.when(c == 1)
            def _():
                do_pass(tables[1], outs[1])

    return prop


_sc_deg = _make_sc_prop(1, const_ones=True)
_sc_prop1 = _make_sc_prop(1)
_sc_prop128 = _make_sc_prop64(two_pass=False)
_sc_prop256 = _make_sc_prop64(two_pass=True)


def _tca_body(d0, d1, x, dis_o, t0a_o, t0b_o):
    deg = d0[...] + d1[...] + 1.0
    dis = lax.rsqrt(deg)
    dis_o[...] = dis
    t0 = x[...] * dis
    t0a_o[...] = t0[:, :64]
    t0b_o[...] = t0[:, 64:]


def _tc_a(d0, d1, x):
    R = 2000
    return pl.pallas_call(
        _tca_body,
        grid=(N // R,),
        in_specs=[
            pl.BlockSpec((R, 1), lambda i: (i, 0)),
            pl.BlockSpec((R, 1), lambda i: (i, 0)),
            pl.BlockSpec((R, 128), lambda i: (i, 0)),
        ],
        out_specs=[
            pl.BlockSpec((R, 1), lambda i: (i, 0)),
            pl.BlockSpec((R, 64), lambda i: (i, 0)),
            pl.BlockSpec((R, 64), lambda i: (i, 0)),
        ],
        out_shape=[
            jax.ShapeDtypeStruct((N, 1), F32),
            jax.ShapeDtypeStruct((NPAD, 64), F32),
            jax.ShapeDtypeStruct((NPAD, 64), F32),
        ],
    )(d0, d1, x)


def _tcb_body(a0, a1, t0a, t0b, dis, W1, b1, o0, o1, o2, o3):
    d = dis[...]
    z = jnp.concatenate(
        [(a0[...] + t0a[...]) * d, (a1[...] + t0b[...]) * d], axis=1)
    h = jnp.maximum(jnp.dot(z, W1[...], preferred_element_type=F32) + b1[...], 0.0)
    t1 = h * d
    o0[...] = t1[:, 0:64]
    o1[...] = t1[:, 64:128]
    o2[...] = t1[:, 128:192]
    o3[...] = t1[:, 192:256]


def _tc_b(a0, a1, t0a, t0b, dis, W1, b1):
    R = 1000
    q = jax.ShapeDtypeStruct((NPAD, 64), F32)
    return pl.pallas_call(
        _tcb_body,
        grid=(N // R,),
        in_specs=[
            pl.BlockSpec((R, 64), lambda i: (i, 0)),
            pl.BlockSpec((R, 64), lambda i: (i, 0)),
            pl.BlockSpec((R, 64), lambda i: (i, 0)),
            pl.BlockSpec((R, 64), lambda i: (i, 0)),
            pl.BlockSpec((R, 1), lambda i: (i, 0)),
            pl.BlockSpec((128, 256), lambda i: (0, 0)),
            pl.BlockSpec((1, 256), lambda i: (0, 0)),
        ],
        out_specs=[pl.BlockSpec((R, 64), lambda i: (i, 0)) for _ in range(4)],
        out_shape=[q, q, q, q],
    )(a0, a1, t0a, t0b, dis, W1, b1)


def _tcc_body(c0, c1, c2, c3, t0, t1, t2, t3r, dis, W2, b2, W3, o):
    d = dis[...]
    z = jnp.concatenate(
        [(c0[...] + t0[...]) * d, (c1[...] + t1[...]) * d,
         (c2[...] + t2[...]) * d, (c3[...] + t3r[...]) * d], axis=1)
    h = jnp.maximum(jnp.dot(z, W2[...], preferred_element_type=F32) + b2[...], 0.0)
    o[...] = jnp.dot(h, W3[...], preferred_element_type=F32) * d


def _tc_c(cq, tq, dis, W2, b2, W3):
    R = 1000
    return pl.pallas_call(
        _tcc_body,
        grid=(N // R,),
        in_specs=(
            [pl.BlockSpec((R, 64), lambda i: (i, 0)) for _ in range(8)]
            + [
                pl.BlockSpec((R, 1), lambda i: (i, 0)),
                pl.BlockSpec((256, 512), lambda i: (0, 0)),
                pl.BlockSpec((1, 512), lambda i: (0, 0)),
                pl.BlockSpec((512, 1), lambda i: (0, 0)),
            ]
        ),
        out_specs=pl.BlockSpec((R, 1), lambda i: (i, 0)),
        out_shape=jax.ShapeDtypeStruct((NPAD, 1), F32),
    )(*cq, *tq, dis, W2, b2, W3)


def _tcd_body(p0, p1, t3, dis, b3, o):
    z = (p0[...] + p1[...] + t3[...]) * dis[...] + b3[...]
    o[...] = 1.0 / (1.0 + jnp.exp(-z))


def _tc_d(p0, p1, t3, dis, b3):
    R = 2000
    return pl.pallas_call(
        _tcd_body,
        grid=(N // R,),
        in_specs=[
            pl.BlockSpec((R, 1), lambda i: (i, 0)),
            pl.BlockSpec((R, 1), lambda i: (i, 0)),
            pl.BlockSpec((R, 1), lambda i: (i, 0)),
            pl.BlockSpec((R, 1), lambda i: (i, 0)),
            pl.BlockSpec((1, 1), lambda i: (0, 0)),
        ],
        out_specs=pl.BlockSpec((R, 1), lambda i: (i, 0)),
        out_shape=jax.ShapeDtypeStruct((N, 1), F32),
    )(p0, p1, t3, dis, b3)


def kernel(x, edge_index, W1, b1, W2, b2, W3, b3):
    src = edge_index[0].astype(jnp.int32)
    dst = edge_index[1].astype(jnp.int32)
    npadrows = NPAD - N
    pad_dst = N + (jnp.arange(EPAD - E, dtype=jnp.int32) % npadrows)
    src_pad = jnp.concatenate([src, jnp.zeros((EPAD - E,), jnp.int32)])
    dst_pad = jnp.concatenate([dst, pad_dst])
    src2d = src_pad.reshape(EPAD // G, G)
    dst2d = dst_pad.reshape(EPAD // G, G)
    src2d64 = src_pad.reshape(EPAD // 64, 64)
    dst2d64 = dst_pad.reshape(EPAD // 64, 64)

    d0, d1 = _sc_deg(dst2d)
    dis, t0a, t0b = _tc_a(d0.reshape(NPAD, 1), d1.reshape(NPAD, 1), x)
    a0, a1 = _sc_prop128(t0a, t0b, src2d64, dst2d64)
    t1q = _tc_b(a0, a1, t0a, t0b, dis, W1, b1.reshape(1, -1))
    cq = _sc_prop256(*t1q, src2d64, dst2d64)
    t3 = _tc_c(cq, t1q, dis, W2, b2.reshape(1, -1), W3)
    p0, p1 = _sc_prop1(t3.reshape(NPAD), src2d, dst2d)
    return _tc_d(p0.reshape(NPAD, 1), p1.reshape(NPAD, 1), t3, dis,
                 b3.reshape(1, 1))
